# Initial kernel scaffold; baseline (speedup 1.0000x reference)
import jax
import jax.numpy as jnp
from jax.experimental import pallas as pl


def kernel(x, W, b):
    raise NotImplementedError("write your pallas kernel here")


# speedup vs baseline: 167.3489x; 167.3489x over previous
import functools
import math

import jax
import jax.numpy as jnp
from jax.experimental import pallas as pl
from jax.experimental.pallas import tpu as pltpu

_T = math.e
_AR = 0.7


def _body(x_ref, w_ref, b_ref, o_ref, *, na):
    l = jax.lax.dot_general(
        x_ref[...], w_ref[...],
        (((1,), (1,)), ((), ())),
        preferred_element_type=jnp.float32,
        precision=jax.lax.Precision.DEFAULT,
    )
    scaled = (l + b_ref[...]) * (1.0 / _T)
    m = jnp.max(scaled, axis=1, keepdims=True)
    e = jnp.exp(scaled - m)
    esum = jnp.sum(e, axis=1, keepdims=True)

    key = jax.lax.bitcast_convert_type(e, jnp.int32)

    def step(i, prefix):
        trial = prefix | (jnp.int32(1) << (29 - i))
        cnt = jnp.sum((key >= trial).astype(jnp.int32), axis=1, keepdims=True)
        return jnp.where(cnt >= na, trial, prefix)

    thr = jax.lax.fori_loop(
        0, 30, step, jnp.zeros((e.shape[0], 1), jnp.int32))

    kept = key >= thr
    s = jnp.sum(jnp.where(kept, e, 0.0), axis=1, keepdims=True)
    recip = 1.0 / (s + 1e-8 * esum)
    o_ref[...] = jnp.where(kept, e * recip, 0.0)


def kernel(x, W, b):
    rows, d = x.shape
    n = W.shape[0]
    na = max(1, int(n * _AR))
    bm = 256
    while rows % bm:
        bm //= 2
    grid = (rows // bm,)
    b2 = b.reshape(1, n)
    return pl.pallas_call(
        functools.partial(_body, na=na),
        grid=grid,
        in_specs=[
            pl.BlockSpec((bm, d), lambda i: (i, 0)),
            pl.BlockSpec((n, d), lambda i: (0, 0)),
            pl.BlockSpec((1, n), lambda i: (0, 0)),
        ],
        out_specs=pl.BlockSpec((bm, n), lambda i: (i, 0)),
        out_shape=jax.ShapeDtypeStruct((rows, n), jnp.float32),
        compiler_params=pltpu.CompilerParams(
            dimension_semantics=("parallel",),
        ),
    )(x, W, b2)
